# Initial kernel scaffold; baseline (speedup 1.0000x reference)
import jax
import jax.numpy as jnp
from jax.experimental import pallas as pl


def kernel(x, params, kernals):
    raise NotImplementedError("write your pallas kernel here")


# speedup vs baseline: 5.9595x; 5.9595x over previous
import functools

import jax
import jax.numpy as jnp
from jax.experimental import pallas as pl
from jax.experimental.pallas import tpu as pltpu
from jax.experimental.pallas import tpu_sc as plsc


def _knn_body(pts_ref, q_ref, idx_ref, *, k, n, bq):
    pT = pts_ref[0]
    qT = q_ref[0]
    q = qT.T
    s = 2.0 * jnp.dot(q, pT, preferred_element_type=jnp.float32)
    qq = jnp.sum(q * q, axis=1, keepdims=True)
    pp = jnp.sum(pT * pT, axis=0, keepdims=True)
    s = s - qq - pp
    lane = jax.lax.broadcasted_iota(jnp.int32, (bq, n), 1)
    cols = []
    for _ in range(k):
        m = jnp.max(s, axis=1, keepdims=True)
        am = jnp.min(jnp.where(s >= m, lane, n), axis=1, keepdims=True)
        cols.append(am)
        s = jnp.where(lane == am, -jnp.inf, s)
    idx = jnp.concatenate(cols, axis=1)
    b = pl.program_id(0)
    idx_ref[0] = (idx + b * n).T


def _knn(pts, q_len, k):
    B, _, n = pts.shape
    bq = min(256, q_len)
    return pl.pallas_call(
        functools.partial(_knn_body, k=k, n=n, bq=bq),
        grid=(B, q_len // bq),
        in_specs=[
            pl.BlockSpec((1, 3, n), lambda b, i: (b, 0, 0)),
            pl.BlockSpec((1, 3, bq), lambda b, i: (b, 0, i)),
        ],
        out_specs=pl.BlockSpec((1, k, bq), lambda b, i: (b, 0, i)),
        out_shape=jax.ShapeDtypeStruct((B, k, q_len), jnp.int32),
    )(pts, pts)


def _sc_gather(table, idx_flat, width):
    num = idx_flat.shape[0]
    idx2 = idx_flat.reshape(1, num)
    mesh = plsc.VectorSubcoreMesh(core_axis_name="core",
                                  subcore_axis_name="subcore")
    window = 128

    @pl.kernel(out_type=jax.ShapeDtypeStruct((num, width), table.dtype),
               mesh=mesh)
    def gk(tab_hbm, i_hbm, o_hbm):
        def body(i_vmem, o_vmem):
            pltpu.sync_copy(tab_hbm.at[i_vmem.at[0]], o_vmem)

        pltpu.emit_pipeline(
            body,
            grid=(num // window,),
            in_specs=[pl.BlockSpec((1, window), index_map=lambda i: (0, i))],
            out_specs=[pl.BlockSpec((window, width),
                                    index_map=lambda i: (i, 0))],
            core_axis_name="subcore",
            dimension_semantics=(pltpu.PARALLEL,),
        )(i_hbm, o_hbm)

    return gk(table, idx2)


def _gelu(x):
    return 0.5 * x * (1.0 + jax.lax.erf(x * 0.7071067811865476))


def _paiconv_body(g_ref, mlpw_ref, mlpb_ref, kerT_ref, linw_ref, linb_ref,
                  out_ref, stats_ref, *, k, rb, f, foff, ks):
    g = g_ref[0]
    G = g.reshape(k * rb, g.shape[-1])
    GT = G.T
    featT = GT[foff:foff + f]
    xT = GT[0:3]
    x0 = xT[:, 0:rb]
    x0rep = jnp.concatenate([x0] * k, axis=1)
    xrel = xT - x0rep
    dis = jnp.sqrt(jnp.sum(xrel * xrel, axis=0, keepdims=True) + 1e-12)
    x7 = jnp.concatenate([x0rep, xrel, dis], axis=0)
    xf = jnp.dot(mlpw_ref[...], x7,
                 preferred_element_type=jnp.float32) + mlpb_ref[...]

    adj = jnp.dot(kerT_ref[...], xrel, preferred_element_type=jnp.float32)
    rower = jax.lax.broadcasted_iota(jnp.int32, (ks, k * rb), 0)
    laner = jax.lax.broadcasted_iota(jnp.int32, (ks, k * rb), 1)
    adj = adj + jnp.where((rower == 0) & (laner < rb), 1.0, 0.0)
    adj = jnp.maximum(adj, 0.0)

    def jsum(a):
        r = a[:, 0:rb]
        for j in range(1, k):
            r = r + a[:, j * rb:(j + 1) * rb]
        return r

    def jtile(s):
        return jnp.concatenate([s] * k, axis=1)

    a1 = adj / jtile(jsum(adj) + 1e-6)
    t = a1 * a1
    af = t / jtile(jsum(t) + 1e-6)
    af = jnp.where(af > 0.1, af, 0.0)

    chT = jnp.concatenate([featT, xf], axis=0)
    chT = chT.astype(jnp.bfloat16).astype(jnp.float32)
    afb = af.astype(jnp.bfloat16).astype(jnp.float32)
    acc = jnp.zeros((2 * f, ks, rb), jnp.float32)
    for j in range(k):
        cj = chT[:, j * rb:(j + 1) * rb]
        aj = afb[:, j * rb:(j + 1) * rb]
        acc = acc + cj[:, None, :] * aj[None, :, :]
    accf = acc.reshape(2 * f * ks, rb)
    outT = jnp.dot(linw_ref[...], accf,
                   preferred_element_type=jnp.float32) + linb_ref[...]
    out_c = outT.shape[0]
    opad = out_ref.shape[-1]
    o = outT.T
    if opad > out_c:
        o = jnp.concatenate(
            [o, jnp.zeros((rb, opad - out_c), jnp.float32)], axis=1)
    out_ref[0] = o

    first = (pl.program_id(0) == 0) & (pl.program_id(1) == 0)

    @pl.when(first)
    def _():
        stats_ref[...] = jnp.zeros_like(stats_ref)

    stats_ref[:, 0:1] += jnp.sum(outT, axis=1, keepdims=True)
    stats_ref[:, 1:2] += jnp.sum(outT * outT, axis=1, keepdims=True)


def _paiconv(g4, p, kernals, *, f, foff):
    B, k, n, w = g4.shape
    out_c = p['lin_w'].shape[0]
    opad = -(-out_c // 128) * 128
    ks = kernals.shape[1]
    rb = min(512, n)
    out, stats = pl.pallas_call(
        functools.partial(_paiconv_body, k=k, rb=rb, f=f, foff=foff, ks=ks),
        grid=(B, n // rb),
        in_specs=[
            pl.BlockSpec((1, k, rb, w), lambda b, i: (b, 0, i, 0)),
            pl.BlockSpec((f, 7), lambda b, i: (0, 0)),
            pl.BlockSpec((f, 1), lambda b, i: (0, 0)),
            pl.BlockSpec((ks, 3), lambda b, i: (0, 0)),
            pl.BlockSpec((out_c, 2 * f * ks), lambda b, i: (0, 0)),
            pl.BlockSpec((out_c, 1), lambda b, i: (0, 0)),
        ],
        out_specs=[
            pl.BlockSpec((1, rb, opad), lambda b, i: (b, i, 0)),
            pl.BlockSpec((out_c, 2), lambda b, i: (0, 0)),
        ],
        out_shape=[
            jax.ShapeDtypeStruct((B, n, opad), jnp.float32),
            jax.ShapeDtypeStruct((out_c, 2), jnp.float32),
        ],
    )(g4, p['mlp_w'], p['mlp_b'].reshape(f, 1), kernals.T,
      p['lin_w'], p['lin_b'].reshape(out_c, 1))
    return out, stats


def _pool_body(gf_ref, stats_ref, gb_ref, xsub_ref, fpool_ref, table_ref, *,
               kq, rb, f, cnt, wout):
    stats = stats_ref[...]
    m = stats[:, 0:1] / cnt
    v = stats[:, 1:2] / cnt - m * m
    scale = gb_ref[:, 0:1] / jnp.sqrt(v + 1e-5)
    sclT = scale.T
    mT = m.T
    bT = gb_ref[:, 1:2].T
    g = gf_ref[0]
    mx = None
    for j in range(kq):
        y = _gelu((g[j, :, 0:f] - mT) * sclT + bT)
        mx = y if mx is None else jnp.maximum(mx, y)
    fpool_ref[0] = mx.T
    xs = xsub_ref[0].T
    pad = wout - 3 - f
    parts = [xs, mx]
    if pad:
        parts.append(jnp.zeros((rb, pad), jnp.float32))
    table_ref[0] = jnp.concatenate(parts, axis=1)


def _pool(gf4, stats, gb, x, *, cnt, wout):
    B, kq, np_, w_in = gf4.shape
    f = stats.shape[0]
    rb = min(512, np_)
    fpool, table = pl.pallas_call(
        functools.partial(_pool_body, kq=kq, rb=rb, f=f, cnt=cnt, wout=wout),
        grid=(B, np_ // rb),
        in_specs=[
            pl.BlockSpec((1, kq, rb, w_in), lambda b, i: (b, 0, i, 0)),
            pl.BlockSpec((f, 2), lambda b, i: (0, 0)),
            pl.BlockSpec((f, 2), lambda b, i: (0, 0)),
            pl.BlockSpec((1, 3, rb), lambda b, i: (b, 0, i)),
        ],
        out_specs=[
            pl.BlockSpec((1, f, rb), lambda b, i: (b, 0, i)),
            pl.BlockSpec((1, rb, wout), lambda b, i: (b, i, 0)),
        ],
        out_shape=[
            jax.ShapeDtypeStruct((B, f, np_), jnp.float32),
            jax.ShapeDtypeStruct((B, np_, wout), jnp.float32),
        ],
    )(gf4, stats, gb, x)
    return fpool, table


def _head_body(xc_ref, w5_ref, gb5_ref, w1_ref, gb6_ref, w2_ref, b2_ref,
               gb7_ref, w3_ref, b3_ref, out_ref, *, B, nsub):
    def bn_cols(x, gb):
        m = jnp.mean(x, axis=1, keepdims=True)
        v = jnp.mean((x - m) ** 2, axis=1, keepdims=True)
        return gb[:, 0:1] * (x - m) / jnp.sqrt(v + 1e-5) + gb[:, 1:2]

    Y = jnp.dot(w5_ref[...], xc_ref[...],
                preferred_element_type=jnp.float32)
    Y = _gelu(bn_cols(Y, gb5_ref[...]))
    mxs, mns = [], []
    for b in range(B):
        blk = Y[:, b * nsub:(b + 1) * nsub]
        mxs.append(jnp.max(blk, axis=1, keepdims=True))
        mns.append(jnp.mean(blk, axis=1, keepdims=True))
    h = jnp.concatenate([jnp.concatenate(mxs, axis=1),
                         jnp.concatenate(mns, axis=1)], axis=0)
    H = jnp.dot(w1_ref[...], h, preferred_element_type=jnp.float32)
    H = _gelu(bn_cols(H, gb6_ref[...]))
    H = jnp.dot(w2_ref[...], H, preferred_element_type=jnp.float32) \
        + b2_ref[...]
    H = _gelu(bn_cols(H, gb7_ref[...]))
    H = jnp.dot(w3_ref[...], H, preferred_element_type=jnp.float32) \
        + b3_ref[...]
    out_ref[...] = H.T


def _head(xc2, params, *, B, nsub):
    def gb(gk, bk):
        return jnp.stack([params[gk], params[bk]], axis=1)

    return pl.pallas_call(
        functools.partial(_head_body, B=B, nsub=nsub),
        out_shape=jax.ShapeDtypeStruct((B, 40), jnp.float32),
    )(xc2, params['conv5_w'], gb('bn5_g', 'bn5_b'),
      params['lin1_w'], gb('bn6_g', 'bn6_b'),
      params['lin2_w'], params['lin2_b'].reshape(-1, 1),
      gb('bn7_g', 'bn7_b'),
      params['lin3_w'], params['lin3_b'].reshape(-1, 1))


def _stack_gb(p):
    return jnp.stack([p['bn_g'], p['bn_b']], axis=1)


def kernel(x, params, kernals):
    B, _, N = x.shape

    xtr = jnp.transpose(x, (0, 2, 1)).reshape(B * N, 3)
    table1 = jnp.concatenate(
        [xtr, jnp.zeros((B * N, 125), jnp.float32)], axis=1)
    idx1 = _knn(x, N, 20)
    g1 = _sc_gather(table1, idx1.reshape(-1), 128).reshape(B, 20, N, 128)
    out1, st1 = _paiconv(g1, params['c1'], kernals, f=3, foff=0)

    np1 = N // 4
    sidx1 = _knn(x, np1, 20)
    gf1 = _sc_gather(out1.reshape(B * N, 128),
                     sidx1.reshape(-1), 128).reshape(B, 20, np1, 128)
    xs1 = x[:, :, :np1]
    f1, table2 = _pool(gf1, st1, _stack_gb(params['c1']), xs1,
                       cnt=B * N, wout=128)
    x1 = f1[:, :, :N // 32]

    idx2 = _knn(xs1, np1, 20)
    g2 = _sc_gather(table2.reshape(B * np1, 128),
                    idx2.reshape(-1), 128).reshape(B, 20, np1, 128)
    out2, st2 = _paiconv(g2, params['c2'], kernals, f=64, foff=3)

    np2 = N // 8
    sidx2 = _knn(xs1, np2, 20)
    gf2 = _sc_gather(out2.reshape(B * np1, 128),
                     sidx2.reshape(-1), 128).reshape(B, 20, np2, 128)
    xs2 = xs1[:, :, :np2]
    f2, table3 = _pool(gf2, st2, _stack_gb(params['c2']), xs2,
                       cnt=B * np1, wout=128)
    x2 = f2[:, :, :N // 32]

    idx3 = _knn(xs2, np2, 20)
    g3 = _sc_gather(table3.reshape(B * np2, 128),
                    idx3.reshape(-1), 128).reshape(B, 20, np2, 128)
    out3, st3 = _paiconv(g3, params['c3'], kernals, f=64, foff=3)

    np3 = N // 16
    sidx3 = _knn(xs2, np3, 20)
    gf3 = _sc_gather(out3.reshape(B * np2, 128),
                     sidx3.reshape(-1), 128).reshape(B, 20, np3, 128)
    xs3 = xs2[:, :, :np3]
    f3, table4 = _pool(gf3, st3, _stack_gb(params['c3']), xs3,
                       cnt=B * np2, wout=256)
    x3 = f3[:, :, :N // 32]

    idx4 = _knn(xs3, np3, 10)
    g4 = _sc_gather(table4.reshape(B * np3, 256),
                    idx4.reshape(-1), 256).reshape(B, 10, np3, 256)
    out4, st4 = _paiconv(g4, params['c4'], kernals, f=128, foff=3)

    np4 = N // 32
    sidx4 = _knn(xs3, np3, 20)[:, :, :np4]
    gf4 = _sc_gather(out4.reshape(B * np3, 256),
                     sidx4.reshape(-1), 256).reshape(B, 20, np4, 256)
    xs4 = xs3[:, :, :np4]
    f4, _ = _pool(gf4, st4, _stack_gb(params['c4']), xs4,
                  cnt=B * np3, wout=384)
    x4 = f4

    xc = jnp.concatenate([x1, x2, x3, x4], axis=1)
    xc2 = jnp.transpose(xc, (1, 0, 2)).reshape(512, B * np4)
    return _head(xc2, params, B=B, nsub=np4)
